# baseline (speedup 1.0000x reference)
import functools
import math

import jax
import jax.numpy as jnp
from jax import lax
from jax.experimental import pallas as pl
from jax.experimental.pallas import tpu as pltpu
from jax.experimental.pallas import tpu_sc as plsc

N = 100000
E = 3200000
EPS = 1e-5
K0 = math.ceil(0.8 * N)
K1 = math.ceil(0.6 * K0)
K2 = math.ceil(0.4 * K1)

NC, NS, L = 2, 16, 16
NW = NC * NS
EPT = E // NW
CE = 2000
NBITS = 3200
C = 10240
NCHUNK = 10
AGG_ROWS = C * NCHUNK
G = 1024
NGB = G // 128

@functools.lru_cache(maxsize=None)
def _sc_kernels():
  _mesh = plsc.VectorSubcoreMesh(core_axis_name="c", subcore_axis_name="s",
                                 num_cores=NC, num_subcores=NS)
  _CP = pltpu.CompilerParams(needs_layout_passes=False,
                             use_tc_tiling_on_sc=False)

  @functools.partial(
    pl.kernel,
      out_type=(jax.ShapeDtypeStruct((E,), jnp.int32),
                jax.ShapeDtypeStruct((NW, N), jnp.float32)),
      mesh=_mesh, compiler_params=_CP,
      scratch_types=[
          pltpu.VMEM((NBITS,), jnp.int32),
          pltpu.VMEM((N,), jnp.float32),
          pltpu.VMEM((CE,), jnp.int32),
          pltpu.VMEM((CE,), jnp.int32),
          pltpu.VMEM((CE,), jnp.int32),
      ],
  )
  def _sc_remap_deg(src_hbm, dst_hbm, bits_hbm, dstn_hbm, degp_hbm,
                    bits_v, deg_v, src_v, dst_v, dn_v):
      wid = lax.axis_index("s") * NC + lax.axis_index("c")
      base = wid * EPT
      pltpu.sync_copy(bits_hbm, bits_v)

      def zero(i):
          deg_v[pl.ds(i * L, L)] = jnp.zeros((L,), jnp.float32)

      pl.loop(0, NPAD // L)(zero)

      def chunk(ci):
          off = base + ci * CE
          pltpu.sync_copy(src_hbm.at[pl.ds(off, CE)], src_v)
          pltpu.sync_copy(dst_hbm.at[pl.ds(off, CE)], dst_v)

          def grp(g):
              s16 = src_v[pl.ds(g * L, L)]
              d16 = dst_v[pl.ds(g * L, L)]
              dc = jnp.maximum(d16, 0)
              wi_s = (jnp.left_shift(jnp.right_shift(s16, 12), 7)
                      | jnp.bitwise_and(s16, 127))
              wi_d = (jnp.left_shift(jnp.right_shift(dc, 12), 7)
                      | jnp.bitwise_and(dc, 127))
              ws = plsc.load_gather(bits_v, [wi_s])
              wd = plsc.load_gather(bits_v, [wi_d])
              bs = jnp.right_shift(
                  ws, jnp.bitwise_and(jnp.right_shift(s16, 7), 31)) & 1
              bd = jnp.right_shift(
                  wd, jnp.bitwise_and(jnp.right_shift(dc, 7), 31)) & 1
              ok = (bs & bd & jnp.where(d16 >= 0, 1, 0)) == 1
              dn_v[pl.ds(g * L, L)] = jnp.where(ok, d16, -1)
              plsc.addupdate_scatter(deg_v, [dc], jnp.ones((L,), jnp.float32),
                                     mask=ok)

          pl.loop(0, CE // L)(grp)
          pltpu.sync_copy(dn_v, dstn_hbm.at[pl.ds(off, CE)])

      pl.loop(0, EPT // CE)(chunk)
      pltpu.sync_copy(deg_v, degp_hbm.at[wid])


  @functools.partial(
      pl.kernel,
      out_type=jax.ShapeDtypeStruct((AGG_ROWS, 64), jnp.float32),
      mesh=_mesh, compiler_params=_CP,
      scratch_types=[
          pltpu.VMEM((CE,), jnp.int32),
          pltpu.VMEM((CE,), jnp.int32),
          pltpu.VMEM((G + L,), jnp.int32),
          pltpu.VMEM((G + L,), jnp.int32),
          pltpu.VMEM((NGB, 128), jnp.int32),
          pltpu.VMEM((NGB, 128), jnp.int32),
          pltpu.VMEM((G, 64), jnp.float32),
          pltpu.VMEM((128, 64), jnp.float32),
          pltpu.VMEM_SHARED((C + L, 64), jnp.float32),
          pltpu.SemaphoreType.DMA,
      ],
  )
  def _sc_msg(hh2_hbm, src_hbm, dstn_hbm, agg_hbm, src_v, dst_v, cg_v, cs_v,
              g2_v, s2_v, rows_v, zb_v, acc_sh, sem):
      cid = lax.axis_index("c")
      sid = lax.axis_index("s")
      base = sid * (E // NS)

      def zrow(r):
          def zcol(q):
              zb_v[r, pl.ds(q * L, L)] = jnp.zeros((L,), jnp.float32)

          pl.loop(0, 64 // L)(zcol)

      pl.loop(0, 128)(zrow)

      def flush_blocks(nb):
          def cp(b):
              def cpl(q):
                  g2_v[b, pl.ds(q * L, L)] = cg_v[pl.ds(b * 128 + q * L, L)]
                  s2_v[b, pl.ds(q * L, L)] = cs_v[pl.ds(b * 128 + q * L, L)]

              pl.loop(0, 128 // L)(cpl)

          pl.loop(0, nb)(cp)

          def dma(b):
              pltpu.async_copy(hh2_hbm.at[g2_v.at[b]],
                               rows_v.at[pl.ds(b * 128, 128)], sem).wait()

          pl.loop(0, nb)(dma)

          def sca(b):
              pltpu.sync_copy(rows_v.at[pl.ds(b * 128, 128)],
                              acc_sh.at[s2_v.at[b]], add=True)

          pl.loop(0, nb)(sca)

      for j in range(NCHUNK // NC):
          chunk_lo = (cid + NC * j) * C
          z0 = sid * ((C + L) // NS)

          def zchunk(t):
              pltpu.sync_copy(zb_v, acc_sh.at[pl.ds(z0 + t * 128, 128)])

          pl.loop(0, 5)(zchunk)
          pltpu.sync_copy(zb_v.at[pl.ds(0, 1)],
                          acc_sh.at[pl.ds(z0 + 640, 1)])
          plsc.subcore_barrier()

          def ce_chunk(ci, off):
              eoff = base + ci * CE
              pltpu.sync_copy(src_hbm.at[pl.ds(eoff, CE)], src_v)
              pltpu.sync_copy(dstn_hbm.at[pl.ds(eoff, CE)], dst_v)

              def grp(g, off):
                  s16 = src_v[pl.ds(g * L, L)]
                  d16 = dst_v[pl.ds(g * L, L)]
                  rel = d16 - chunk_lo
                  m = (rel >= 0) & (rel < C)
                  plsc.store_compressed(cg_v.at[pl.ds(off, L)], s16, mask=m)
                  plsc.store_compressed(cs_v.at[pl.ds(off, L)], rel, mask=m)
                  off2 = off + plsc.all_reduce_population_count(m)[0]

                  def do_flush():
                      flush_blocks(NGB)
                      cg_v[pl.ds(0, L)] = cg_v[pl.ds(G, L)]
                      cs_v[pl.ds(0, L)] = cs_v[pl.ds(G, L)]

                  pl.when(off2 >= G)(do_flush)
                  return jnp.where(off2 >= G, off2 - G, off2)

              return pl.loop(0, CE // L, init_carry=off)(grp)

          off = pl.loop(0, (E // NS) // CE, init_carry=0)(ce_chunk)

          nb = (off + 127) // 128

          def pad(p):
              cg_v[pl.ds(off + p * L, L)] = jnp.zeros((L,), jnp.int32)
              cs_v[pl.ds(off + p * L, L)] = jnp.full((L,), C, jnp.int32)

          pl.loop(0, (nb * 128 - off + L - 1) // L)(pad)
          flush_blocks(nb)
          plsc.subcore_barrier()

          w0 = sid * (C // NS)

          def wchunk(t):
              pltpu.sync_copy(acc_sh.at[pl.ds(w0 + t * 128, 128)],
                              agg_hbm.at[pl.ds(chunk_lo + w0 + t * 128, 128)])

          pl.loop(0, 5)(wchunk)
          plsc.subcore_barrier()


  return _sc_remap_deg, _sc_msg


NPAD = 102400
NR = 25
RB = 4096
NBW = 3200
TOTK = K0 + K1 + K2
IMIN = -2147483648


def _bn_scale(g):
    return g / math.sqrt(1.0 + EPS)


def _sortable(score):
    b = lax.bitcast_convert_type(score, jnp.int32)
    return b ^ (jnp.right_shift(b, 31) & jnp.int32(0x7FFFFFFF))


def _tc_pre_kernel(x_ref, degp_ref, w1_ref, b1_ref, g1_ref, h1_ref, w2_ref,
                   hh2_ref, dinv_ref):
    deg = 1.0 + jnp.sum(degp_ref[...], axis=0)
    dinv = lax.rsqrt(deg)
    h = x_ref[...] @ w1_ref[...] + b1_ref[...]
    h = h * _bn_scale(1.0) * g1_ref[...] + h1_ref[...]
    h = jnp.maximum(h, 0.0)
    hh2_ref[...] = (h @ w2_ref[...]) * dinv[:, None]
    dinv_ref[...] = dinv.reshape(1, 1, RB)


def _tc_pre(x_in, degp, p):
    ci = x_in.shape[1]
    return pl.pallas_call(
        _tc_pre_kernel,
        grid=(NR,),
        in_specs=[
            pl.BlockSpec((RB, ci), lambda i: (i, 0)),
            pl.BlockSpec((NW, RB), lambda i: (0, i)),
            pl.BlockSpec((ci, 64), lambda i: (0, 0)),
            pl.BlockSpec((1, 64), lambda i: (0, 0)),
            pl.BlockSpec((1, 64), lambda i: (0, 0)),
            pl.BlockSpec((1, 64), lambda i: (0, 0)),
            pl.BlockSpec((64, 64), lambda i: (0, 0)),
        ],
        out_specs=[
            pl.BlockSpec((RB, 64), lambda i: (i, 0)),
            pl.BlockSpec((1, 1, RB), lambda i: (i, 0, 0)),
        ],
        out_shape=[
            jax.ShapeDtypeStruct((NPAD, 64), jnp.float32),
            jax.ShapeDtypeStruct((NR, 1, RB), jnp.float32),
        ],
    )(x_in, degp, p['mlp_W'], p['mlp_b'][None, :], p['mlp_bn_g'][None, :],
      p['mlp_bn_b'][None, :], p['conv_W'])


def _tc_post_kernel(agg_ref, hh2_ref, dinv_ref, cb_ref, g2_ref, h2_ref,
                    pv_ref, x3_ref, score_ref):
    dinv = dinv_ref[...].reshape(RB, 1)
    o = (agg_ref[...] + hh2_ref[...]) * dinv + cb_ref[...]
    o = o * _bn_scale(1.0) * g2_ref[...] + h2_ref[...]
    x3 = jnp.maximum(o, 0.0)
    x3_ref[...] = x3
    pv = pv_ref[...]
    pn = lax.rsqrt(jnp.sum(pv * pv))
    score_ref[...] = (x3 @ pv.reshape(64, 1) * pn).reshape(1, 1, RB)


def _tc_post(agg, hh2, dinv, p):
    return pl.pallas_call(
        _tc_post_kernel,
        grid=(NR,),
        in_specs=[
            pl.BlockSpec((RB, 64), lambda i: (i, 0)),
            pl.BlockSpec((RB, 64), lambda i: (i, 0)),
            pl.BlockSpec((1, 1, RB), lambda i: (i, 0, 0)),
            pl.BlockSpec((1, 64), lambda i: (0, 0)),
            pl.BlockSpec((1, 64), lambda i: (0, 0)),
            pl.BlockSpec((1, 64), lambda i: (0, 0)),
            pl.BlockSpec((1, 64), lambda i: (0, 0)),
        ],
        out_specs=[
            pl.BlockSpec((RB, 64), lambda i: (i, 0)),
            pl.BlockSpec((1, 1, RB), lambda i: (i, 0, 0)),
        ],
        out_shape=[
            jax.ShapeDtypeStruct((NPAD, 64), jnp.float32),
            jax.ShapeDtypeStruct((NR, 1, RB), jnp.float32),
        ],
    )(agg, hh2, dinv, p['conv_b'][None, :], p['bn_g'][None, :],
      p['bn_b'][None, :], p['pool_p'][None, :])


def _make_sel1(k):
    def body(score_ref, alive_ref, vn_ref):
        key = jnp.where(alive_ref[...] > 0.0, _sortable(score_ref[...]),
                        jnp.int32(IMIN))
        idx = (lax.broadcasted_iota(jnp.int32, (NR, 1, RB), 0) * RB
               + lax.broadcasted_iota(jnp.int32, (NR, 1, RB), 2))

        def bit_step(i, tu):
            cand = tu | (jnp.int32(1) << (31 - i))
            th = cand ^ jnp.int32(IMIN)
            cnt = jnp.sum(jnp.where(key >= th, 1, 0))
            return jnp.where(cnt >= k, cand, tu)

        tu = lax.fori_loop(0, 32, bit_step, jnp.int32(0))
        v = tu ^ jnp.int32(IMIN)
        gt = key > v
        eq = key == v
        need = k - jnp.sum(jnp.where(gt, 1, 0))

        def idx_step(i, t):
            cand = t + (jnp.int32(1) << (16 - i))
            cnt = jnp.sum(jnp.where(eq & (idx < cand), 1, 0))
            return jnp.where(cnt < need, cand, t)

        t = lax.fori_loop(0, 17, idx_step, jnp.int32(0))
        lanes = lax.broadcasted_iota(jnp.int32, (1, 128), 1)
        vn_ref[...] = jnp.where(lanes == 0, v,
                                jnp.where(lanes == 1, t + 1, 0))

    return body


def _tc_sel1(score, alivef, k):
    return pl.pallas_call(
        _make_sel1(k),
        out_shape=jax.ShapeDtypeStruct((1, 128), jnp.int32),
    )(score, alivef)


def _tc_sel2_kernel(score_ref, alive_ref, x3_ref, vn_ref,
                    xout_ref, keep_ref, bits_ref, psum_ref):
    i = pl.program_id(0)
    v = vn_ref[0, 0]
    t = vn_ref[0, 1]
    score = score_ref[...].reshape(1, RB)
    key = jnp.where(alive_ref[...].reshape(1, RB) > 0.0, _sortable(score),
                    jnp.int32(IMIN))
    idx = i * RB + lax.broadcasted_iota(jnp.int32, (1, RB), 1)
    keep = (key > v) | ((key == v) & (idx < t))
    keepf = jnp.where(keep, 1.0, 0.0)
    keep_ref[...] = keepf.reshape(1, 1, RB)
    gate = jnp.tanh(score) * keepf
    xout = jnp.where(keepf.reshape(RB, 1) > 0.0,
                     x3_ref[...] * gate.reshape(RB, 1), 0.0)
    xout_ref[...] = xout
    psum_ref[...] = jnp.sum(xout, axis=0).reshape(1, 1, 64)
    ki = jnp.where(keep, 1, 0)
    w = jnp.zeros((1, 128), jnp.int32)
    for j in range(32):
        w = w + ki[:, j * 128:(j + 1) * 128] * jnp.int32(
            (1 << j) if j < 31 else IMIN)
    bits_ref[...] = w.reshape(1, 1, 128)


def _tc_sel2(score, alivef, x3, vn):
    return pl.pallas_call(
        _tc_sel2_kernel,
        grid=(NR,),
        in_specs=[
            pl.BlockSpec((1, 1, RB), lambda i: (i, 0, 0)),
            pl.BlockSpec((1, 1, RB), lambda i: (i, 0, 0)),
            pl.BlockSpec((RB, 64), lambda i: (i, 0)),
            pl.BlockSpec((1, 128), lambda i: (0, 0)),
        ],
        out_specs=[
            pl.BlockSpec((RB, 64), lambda i: (i, 0)),
            pl.BlockSpec((1, 1, RB), lambda i: (i, 0, 0)),
            pl.BlockSpec((1, 1, RB // 32), lambda i: (i, 0, 0)),
            pl.BlockSpec((1, 1, 64), lambda i: (i, 0, 0)),
        ],
        out_shape=[
            jax.ShapeDtypeStruct((NPAD, 64), jnp.float32),
            jax.ShapeDtypeStruct((NR, 1, RB), jnp.float32),
            jax.ShapeDtypeStruct((NR, 1, RB // 32), jnp.int32),
            jax.ShapeDtypeStruct((NR, 1, 64), jnp.float32),
        ],
    )(score, alivef, x3, vn)


def _tc_head_kernel(p0_ref, p1_ref, p2_ref, w1_ref, b1_ref, g1_ref, h1_ref,
                    w2_ref, b2_ref, out_ref):
    tot = (jnp.sum(p0_ref[...], axis=(0, 1)) + jnp.sum(p1_ref[...], axis=(0, 1))
           + jnp.sum(p2_ref[...], axis=(0, 1)))
    xg = (tot / float(TOTK)).reshape(1, 64)
    h = xg @ w1_ref[...] + b1_ref[...]
    h = h * _bn_scale(1.0) * g1_ref[...] + h1_ref[...]
    h = jnp.maximum(h, 0.0)
    out_ref[...] = h @ w2_ref[...] + b2_ref[...]


def _tc_head(p0, p1, p2, d1p, d0p):
    return pl.pallas_call(
        _tc_head_kernel,
        out_shape=jax.ShapeDtypeStruct((1, d0p['W'].shape[1]), jnp.float32),
    )(p0, p1, p2, d1p['W'], d1p['b'][None, :], d1p['bn_g'][None, :],
      d1p['bn_b'][None, :], d0p['W'], d0p['b'][None, :])


def kernel(x, edge_index, params):
    src, dst = edge_index[0], edge_index[1]
    _sc_remap_deg, _sc_msg = _sc_kernels()

    bits = jnp.full((NBITS,), -1, jnp.int32)
    alivef = (jnp.arange(NPAD, dtype=jnp.int32) < N).astype(
        jnp.float32).reshape(NR, 1, RB)
    x_in = jnp.zeros((NPAD, x.shape[1]), x.dtype).at[:N].set(x)
    dst_cur = dst
    psums = []
    for pname, k in (('enc0', K0), ('enc1', K1), ('enc2', K2)):
        p = params[pname]
        dstn, degp = _sc_remap_deg(src, dst_cur, bits)
        hh2, dinv = _tc_pre(x_in, degp, p)
        agg = _sc_msg(hh2, src, dstn)
        x3, score = _tc_post(agg, hh2, dinv, p)
        vn = _tc_sel1(score, alivef, k)
        xout, keepf, bitsw, psum = _tc_sel2(score, alivef, x3, vn)
        bits = bitsw.reshape(NBITS)
        alivef = keepf
        x_in = xout
        dst_cur = dstn
        psums.append(psum)

    return _tc_head(psums[0], psums[1], psums[2],
                    params['dec1'], params['dec0'])


# speedup vs baseline: 1.1158x; 1.1158x over previous
import functools
import math

import jax
import jax.numpy as jnp
from jax import lax
from jax.experimental import pallas as pl
from jax.experimental.pallas import tpu as pltpu
from jax.experimental.pallas import tpu_sc as plsc

N = 100000
E = 3200000
EPS = 1e-5
K0 = math.ceil(0.8 * N)
K1 = math.ceil(0.6 * K0)
K2 = math.ceil(0.4 * K1)

NC, NS, L = 2, 16, 16
NW = NC * NS
EPT = E // NW
CE = 2000
NBITS = 3200
C = 10240
NCHUNK = 10
AGG_ROWS = C * NCHUNK
G = 1024
NGB = G // 128

@functools.lru_cache(maxsize=None)
def _sc_kernels():
  _mesh = plsc.VectorSubcoreMesh(core_axis_name="c", subcore_axis_name="s",
                                 num_cores=NC, num_subcores=NS)
  _CP = pltpu.CompilerParams(needs_layout_passes=False,
                             use_tc_tiling_on_sc=False)

  @functools.partial(
    pl.kernel,
      out_type=(jax.ShapeDtypeStruct((E,), jnp.int32),
                jax.ShapeDtypeStruct((NW, N), jnp.float32)),
      mesh=_mesh, compiler_params=_CP,
      scratch_types=[
          pltpu.VMEM((NBITS,), jnp.int32),
          pltpu.VMEM((N,), jnp.float32),
          pltpu.VMEM((CE,), jnp.int32),
          pltpu.VMEM((CE,), jnp.int32),
          pltpu.VMEM((CE,), jnp.int32),
      ],
  )
  def _sc_remap_deg(src_hbm, dst_hbm, bits_hbm, dstn_hbm, degp_hbm,
                    bits_v, deg_v, src_v, dst_v, dn_v):
      wid = lax.axis_index("s") * NC + lax.axis_index("c")
      base = wid * EPT
      pltpu.sync_copy(bits_hbm, bits_v)

      def zero(i):
          deg_v[pl.ds(i * L, L)] = jnp.zeros((L,), jnp.float32)

      pl.loop(0, NPAD // L)(zero)

      def chunk(ci):
          off = base + ci * CE
          pltpu.sync_copy(src_hbm.at[pl.ds(off, CE)], src_v)
          pltpu.sync_copy(dst_hbm.at[pl.ds(off, CE)], dst_v)

          def grp(g):
              s16 = src_v[pl.ds(g * L, L)]
              d16 = dst_v[pl.ds(g * L, L)]
              dc = jnp.maximum(d16, 0)
              wi_s = (jnp.left_shift(jnp.right_shift(s16, 12), 7)
                      | jnp.bitwise_and(s16, 127))
              wi_d = (jnp.left_shift(jnp.right_shift(dc, 12), 7)
                      | jnp.bitwise_and(dc, 127))
              ws = plsc.load_gather(bits_v, [wi_s])
              wd = plsc.load_gather(bits_v, [wi_d])
              bs = jnp.right_shift(
                  ws, jnp.bitwise_and(jnp.right_shift(s16, 7), 31)) & 1
              bd = jnp.right_shift(
                  wd, jnp.bitwise_and(jnp.right_shift(dc, 7), 31)) & 1
              ok = (bs & bd & jnp.where(d16 >= 0, 1, 0)) == 1
              dn_v[pl.ds(g * L, L)] = jnp.where(ok, d16, -1)
              plsc.addupdate_scatter(deg_v, [dc], jnp.ones((L,), jnp.float32),
                                     mask=ok)

          pl.loop(0, CE // L)(grp)
          pltpu.sync_copy(dn_v, dstn_hbm.at[pl.ds(off, CE)])

      pl.loop(0, EPT // CE)(chunk)
      pltpu.sync_copy(deg_v, degp_hbm.at[wid])


  @functools.partial(
      pl.kernel,
      out_type=jax.ShapeDtypeStruct((AGG_ROWS, 64), jnp.float32),
      mesh=_mesh, compiler_params=_CP,
      scratch_types=[
          pltpu.VMEM((CE,), jnp.int32),
          pltpu.VMEM((CE,), jnp.int32),
          pltpu.VMEM((G + L,), jnp.int32),
          pltpu.VMEM((G + L,), jnp.int32),
          pltpu.VMEM((NGB, 128), jnp.int32),
          pltpu.VMEM((NGB, 128), jnp.int32),
          pltpu.VMEM((G, 64), jnp.float32),
          pltpu.VMEM((128, 64), jnp.float32),
          pltpu.VMEM_SHARED((C + L, 64), jnp.float32),
          pltpu.SemaphoreType.DMA,
      ],
  )
  def _sc_msg(hh2_hbm, src_hbm, dstn_hbm, agg_hbm, src_v, dst_v, cg_v, cs_v,
              g2_v, s2_v, rows_v, zb_v, acc_sh, sem):
      cid = lax.axis_index("c")
      sid = lax.axis_index("s")
      base = sid * (E // NS)

      def zrow(r):
          def zcol(q):
              zb_v[r, pl.ds(q * L, L)] = jnp.zeros((L,), jnp.float32)

          pl.loop(0, 64 // L)(zcol)

      pl.loop(0, 128)(zrow)

      def _cp_idx(nb):
          def cp(b):
              def cpl(q):
                  g2_v[b, pl.ds(q * L, L)] = cg_v[pl.ds(b * 128 + q * L, L)]
                  s2_v[b, pl.ds(q * L, L)] = cs_v[pl.ds(b * 128 + q * L, L)]

              pl.loop(0, 128 // L)(cpl)

          pl.loop(0, nb)(cp)

      def flush_full():
          _cp_idx(NGB)
          descs = [
              pltpu.async_copy(hh2_hbm.at[g2_v.at[b]],
                               rows_v.at[pl.ds(b * 128, 128)], sem)
              for b in range(NGB)
          ]
          for b in range(NGB):
              descs[b].wait()
              pltpu.sync_copy(rows_v.at[pl.ds(b * 128, 128)],
                              acc_sh.at[s2_v.at[b]], add=True)

      def flush_blocks(nb):
          _cp_idx(nb)

          def dma(b):
              pltpu.async_copy(hh2_hbm.at[g2_v.at[b]],
                               rows_v.at[pl.ds(b * 128, 128)], sem).wait()

          pl.loop(0, nb)(dma)

          def sca(b):
              pltpu.sync_copy(rows_v.at[pl.ds(b * 128, 128)],
                              acc_sh.at[s2_v.at[b]], add=True)

          pl.loop(0, nb)(sca)

      for j in range(NCHUNK // NC):
          chunk_lo = (cid + NC * j) * C
          z0 = sid * ((C + L) // NS)

          def zchunk(t):
              pltpu.sync_copy(zb_v, acc_sh.at[pl.ds(z0 + t * 128, 128)])

          pl.loop(0, 5)(zchunk)
          pltpu.sync_copy(zb_v.at[pl.ds(0, 1)],
                          acc_sh.at[pl.ds(z0 + 640, 1)])
          plsc.subcore_barrier()

          def ce_chunk(ci, off):
              eoff = base + ci * CE
              pltpu.sync_copy(src_hbm.at[pl.ds(eoff, CE)], src_v)
              pltpu.sync_copy(dstn_hbm.at[pl.ds(eoff, CE)], dst_v)

              def grp(g, off):
                  s16 = src_v[pl.ds(g * L, L)]
                  d16 = dst_v[pl.ds(g * L, L)]
                  rel = d16 - chunk_lo
                  m = (rel >= 0) & (rel < C)
                  plsc.store_compressed(cg_v.at[pl.ds(off, L)], s16, mask=m)
                  plsc.store_compressed(cs_v.at[pl.ds(off, L)], rel, mask=m)
                  off2 = off + plsc.all_reduce_population_count(m)[0]

                  def do_flush():
                      flush_full()
                      cg_v[pl.ds(0, L)] = cg_v[pl.ds(G, L)]
                      cs_v[pl.ds(0, L)] = cs_v[pl.ds(G, L)]

                  pl.when(off2 >= G)(do_flush)
                  return jnp.where(off2 >= G, off2 - G, off2)

              return pl.loop(0, CE // L, init_carry=off)(grp)

          off = pl.loop(0, (E // NS) // CE, init_carry=0)(ce_chunk)

          nb = (off + 127) // 128

          def pad(p):
              cg_v[pl.ds(off + p * L, L)] = jnp.zeros((L,), jnp.int32)
              cs_v[pl.ds(off + p * L, L)] = jnp.full((L,), C, jnp.int32)

          pl.loop(0, (nb * 128 - off + L - 1) // L)(pad)
          flush_blocks(nb)
          plsc.subcore_barrier()

          w0 = sid * (C // NS)

          def wchunk(t):
              pltpu.sync_copy(acc_sh.at[pl.ds(w0 + t * 128, 128)],
                              agg_hbm.at[pl.ds(chunk_lo + w0 + t * 128, 128)])

          pl.loop(0, 5)(wchunk)
          plsc.subcore_barrier()


  return _sc_remap_deg, _sc_msg


NPAD = 102400
NR = 25
RB = 4096
NBW = 3200
TOTK = K0 + K1 + K2
IMIN = -2147483648


def _bn_scale(g):
    return g / math.sqrt(1.0 + EPS)


def _sortable(score):
    b = lax.bitcast_convert_type(score, jnp.int32)
    return b ^ (jnp.right_shift(b, 31) & jnp.int32(0x7FFFFFFF))


def _tc_pre_kernel(x_ref, degp_ref, w1_ref, b1_ref, g1_ref, h1_ref, w2_ref,
                   hh2_ref, dinv_ref):
    deg = 1.0 + jnp.sum(degp_ref[...], axis=0)
    dinv = lax.rsqrt(deg)
    h = x_ref[...] @ w1_ref[...] + b1_ref[...]
    h = h * _bn_scale(1.0) * g1_ref[...] + h1_ref[...]
    h = jnp.maximum(h, 0.0)
    hh2_ref[...] = (h @ w2_ref[...]) * dinv[:, None]
    dinv_ref[...] = dinv.reshape(1, 1, RB)


def _tc_pre(x_in, degp, p):
    ci = x_in.shape[1]
    return pl.pallas_call(
        _tc_pre_kernel,
        grid=(NR,),
        in_specs=[
            pl.BlockSpec((RB, ci), lambda i: (i, 0)),
            pl.BlockSpec((NW, RB), lambda i: (0, i)),
            pl.BlockSpec((ci, 64), lambda i: (0, 0)),
            pl.BlockSpec((1, 64), lambda i: (0, 0)),
            pl.BlockSpec((1, 64), lambda i: (0, 0)),
            pl.BlockSpec((1, 64), lambda i: (0, 0)),
            pl.BlockSpec((64, 64), lambda i: (0, 0)),
        ],
        out_specs=[
            pl.BlockSpec((RB, 64), lambda i: (i, 0)),
            pl.BlockSpec((1, 1, RB), lambda i: (i, 0, 0)),
        ],
        out_shape=[
            jax.ShapeDtypeStruct((NPAD, 64), jnp.float32),
            jax.ShapeDtypeStruct((NR, 1, RB), jnp.float32),
        ],
    )(x_in, degp, p['mlp_W'], p['mlp_b'][None, :], p['mlp_bn_g'][None, :],
      p['mlp_bn_b'][None, :], p['conv_W'])


def _tc_post_kernel(agg_ref, hh2_ref, dinv_ref, cb_ref, g2_ref, h2_ref,
                    pv_ref, x3_ref, score_ref):
    dinv = dinv_ref[...].reshape(RB, 1)
    o = (agg_ref[...] + hh2_ref[...]) * dinv + cb_ref[...]
    o = o * _bn_scale(1.0) * g2_ref[...] + h2_ref[...]
    x3 = jnp.maximum(o, 0.0)
    x3_ref[...] = x3
    pv = pv_ref[...]
    pn = lax.rsqrt(jnp.sum(pv * pv))
    score_ref[...] = (x3 @ pv.reshape(64, 1) * pn).reshape(1, 1, RB)


def _tc_post(agg, hh2, dinv, p):
    return pl.pallas_call(
        _tc_post_kernel,
        grid=(NR,),
        in_specs=[
            pl.BlockSpec((RB, 64), lambda i: (i, 0)),
            pl.BlockSpec((RB, 64), lambda i: (i, 0)),
            pl.BlockSpec((1, 1, RB), lambda i: (i, 0, 0)),
            pl.BlockSpec((1, 64), lambda i: (0, 0)),
            pl.BlockSpec((1, 64), lambda i: (0, 0)),
            pl.BlockSpec((1, 64), lambda i: (0, 0)),
            pl.BlockSpec((1, 64), lambda i: (0, 0)),
        ],
        out_specs=[
            pl.BlockSpec((RB, 64), lambda i: (i, 0)),
            pl.BlockSpec((1, 1, RB), lambda i: (i, 0, 0)),
        ],
        out_shape=[
            jax.ShapeDtypeStruct((NPAD, 64), jnp.float32),
            jax.ShapeDtypeStruct((NR, 1, RB), jnp.float32),
        ],
    )(agg, hh2, dinv, p['conv_b'][None, :], p['bn_g'][None, :],
      p['bn_b'][None, :], p['pool_p'][None, :])


def _make_sel1(k):
    def body(score_ref, alive_ref, vn_ref):
        key = jnp.where(alive_ref[...] > 0.0, _sortable(score_ref[...]),
                        jnp.int32(IMIN))
        idx = (lax.broadcasted_iota(jnp.int32, (NR, 1, RB), 0) * RB
               + lax.broadcasted_iota(jnp.int32, (NR, 1, RB), 2))

        def bit_step(i, tu):
            cand = tu | (jnp.int32(1) << (31 - i))
            th = cand ^ jnp.int32(IMIN)
            cnt = jnp.sum(jnp.where(key >= th, 1, 0))
            return jnp.where(cnt >= k, cand, tu)

        tu = lax.fori_loop(0, 32, bit_step, jnp.int32(0))
        v = tu ^ jnp.int32(IMIN)
        gt = key > v
        eq = key == v
        need = k - jnp.sum(jnp.where(gt, 1, 0))

        def idx_step(i, t):
            cand = t + (jnp.int32(1) << (16 - i))
            cnt = jnp.sum(jnp.where(eq & (idx < cand), 1, 0))
            return jnp.where(cnt < need, cand, t)

        t = lax.fori_loop(0, 17, idx_step, jnp.int32(0))
        lanes = lax.broadcasted_iota(jnp.int32, (1, 128), 1)
        vn_ref[...] = jnp.where(lanes == 0, v,
                                jnp.where(lanes == 1, t + 1, 0))

    return body


def _tc_sel1(score, alivef, k):
    return pl.pallas_call(
        _make_sel1(k),
        out_shape=jax.ShapeDtypeStruct((1, 128), jnp.int32),
    )(score, alivef)


def _tc_sel2_kernel(score_ref, alive_ref, x3_ref, vn_ref,
                    xout_ref, keep_ref, bits_ref, psum_ref):
    i = pl.program_id(0)
    v = vn_ref[0, 0]
    t = vn_ref[0, 1]
    score = score_ref[...].reshape(1, RB)
    key = jnp.where(alive_ref[...].reshape(1, RB) > 0.0, _sortable(score),
                    jnp.int32(IMIN))
    idx = i * RB + lax.broadcasted_iota(jnp.int32, (1, RB), 1)
    keep = (key > v) | ((key == v) & (idx < t))
    keepf = jnp.where(keep, 1.0, 0.0)
    keep_ref[...] = keepf.reshape(1, 1, RB)
    gate = jnp.tanh(score) * keepf
    xout = jnp.where(keepf.reshape(RB, 1) > 0.0,
                     x3_ref[...] * gate.reshape(RB, 1), 0.0)
    xout_ref[...] = xout
    psum_ref[...] = jnp.sum(xout, axis=0).reshape(1, 1, 64)
    ki = jnp.where(keep, 1, 0)
    w = jnp.zeros((1, 128), jnp.int32)
    for j in range(32):
        w = w + ki[:, j * 128:(j + 1) * 128] * jnp.int32(
            (1 << j) if j < 31 else IMIN)
    bits_ref[...] = w.reshape(1, 1, 128)


def _tc_sel2(score, alivef, x3, vn):
    return pl.pallas_call(
        _tc_sel2_kernel,
        grid=(NR,),
        in_specs=[
            pl.BlockSpec((1, 1, RB), lambda i: (i, 0, 0)),
            pl.BlockSpec((1, 1, RB), lambda i: (i, 0, 0)),
            pl.BlockSpec((RB, 64), lambda i: (i, 0)),
            pl.BlockSpec((1, 128), lambda i: (0, 0)),
        ],
        out_specs=[
            pl.BlockSpec((RB, 64), lambda i: (i, 0)),
            pl.BlockSpec((1, 1, RB), lambda i: (i, 0, 0)),
            pl.BlockSpec((1, 1, RB // 32), lambda i: (i, 0, 0)),
            pl.BlockSpec((1, 1, 64), lambda i: (i, 0, 0)),
        ],
        out_shape=[
            jax.ShapeDtypeStruct((NPAD, 64), jnp.float32),
            jax.ShapeDtypeStruct((NR, 1, RB), jnp.float32),
            jax.ShapeDtypeStruct((NR, 1, RB // 32), jnp.int32),
            jax.ShapeDtypeStruct((NR, 1, 64), jnp.float32),
        ],
    )(score, alivef, x3, vn)


def _tc_head_kernel(p0_ref, p1_ref, p2_ref, w1_ref, b1_ref, g1_ref, h1_ref,
                    w2_ref, b2_ref, out_ref):
    tot = (jnp.sum(p0_ref[...], axis=(0, 1)) + jnp.sum(p1_ref[...], axis=(0, 1))
           + jnp.sum(p2_ref[...], axis=(0, 1)))
    xg = (tot / float(TOTK)).reshape(1, 64)
    h = xg @ w1_ref[...] + b1_ref[...]
    h = h * _bn_scale(1.0) * g1_ref[...] + h1_ref[...]
    h = jnp.maximum(h, 0.0)
    out_ref[...] = h @ w2_ref[...] + b2_ref[...]


def _tc_head(p0, p1, p2, d1p, d0p):
    return pl.pallas_call(
        _tc_head_kernel,
        out_shape=jax.ShapeDtypeStruct((1, d0p['W'].shape[1]), jnp.float32),
    )(p0, p1, p2, d1p['W'], d1p['b'][None, :], d1p['bn_g'][None, :],
      d1p['bn_b'][None, :], d0p['W'], d0p['b'][None, :])


def kernel(x, edge_index, params):
    src, dst = edge_index[0], edge_index[1]
    _sc_remap_deg, _sc_msg = _sc_kernels()

    bits = jnp.full((NBITS,), -1, jnp.int32)
    alivef = (jnp.arange(NPAD, dtype=jnp.int32) < N).astype(
        jnp.float32).reshape(NR, 1, RB)
    x_in = jnp.zeros((NPAD, x.shape[1]), x.dtype).at[:N].set(x)
    dst_cur = dst
    psums = []
    for pname, k in (('enc0', K0), ('enc1', K1), ('enc2', K2)):
        p = params[pname]
        dstn, degp = _sc_remap_deg(src, dst_cur, bits)
        hh2, dinv = _tc_pre(x_in, degp, p)
        agg = _sc_msg(hh2, src, dstn)
        x3, score = _tc_post(agg, hh2, dinv, p)
        vn = _tc_sel1(score, alivef, k)
        xout, keepf, bitsw, psum = _tc_sel2(score, alivef, x3, vn)
        bits = bitsw.reshape(NBITS)
        alivef = keepf
        x_in = xout
        dst_cur = dstn
        psums.append(psum)

    return _tc_head(psums[0], psums[1], psums[2],
                    params['dec1'], params['dec0'])
